# Initial kernel scaffold; baseline (speedup 1.0000x reference)
import jax
import jax.numpy as jnp
from jax.experimental import pallas as pl


def kernel(llm_logits, slm_logits, W1, b1, W2, b2, W3, b3):
    raise NotImplementedError("write your pallas kernel here")


# speedup vs baseline: 44.5904x; 44.5904x over previous
import jax
import jax.numpy as jnp
from jax.experimental import pallas as pl
from jax.experimental.pallas import tpu as pltpu

_K = 10
_B = 128
_V = 32768
_RG = 16
_LANES = 128
_CHUNK = 8192
_NC = _V // _CHUNK
_NSL = _CHUNK // _LANES
_NG = _B // _RG
_NEG = -3.0e38


def _insert_chunk(x_ref, tiles_ref):
    tiles = [tiles_ref[:, k * _LANES:(k + 1) * _LANES] for k in range(_K)]
    for j in range(_NSL):
        v = x_ref[:, j * _LANES:(j + 1) * _LANES]
        for k in range(_K):
            t = tiles[k]
            hi = jnp.maximum(t, v)
            v = jnp.minimum(t, v)
            tiles[k] = hi
    for k in range(_K):
        tiles_ref[:, k * _LANES:(k + 1) * _LANES] = tiles[k]


def _merge_topk(tiles_ref):
    cand = tiles_ref[...]
    idx = jax.lax.broadcasted_iota(jnp.int32, cand.shape, 1)
    outs = []
    for _ in range(_K):
        m = jnp.max(cand, axis=1, keepdims=True)
        eq = cand == m
        pos = jnp.min(jnp.where(eq, idx, _K * _LANES), axis=1, keepdims=True)
        cand = jnp.where(idx == pos, _NEG, cand)
        outs.append(m)
    return jnp.concatenate(outs, axis=1)


def _mlp(c, w1t_ref, b1_ref, w2t_ref, b2_ref, w3t_ref, b3_ref):
    z1 = jnp.dot(c, w1t_ref[...], preferred_element_type=jnp.float32) + b1_ref[...]
    h1 = jnp.maximum(z1, 0.0)
    z2 = jnp.dot(h1, w2t_ref[...], preferred_element_type=jnp.float32) + b2_ref[...]
    h2 = jnp.maximum(z2, 0.0)
    z3 = jnp.dot(h2, w3t_ref[...], preferred_element_type=jnp.float32) + b3_ref[...]
    raw = jax.nn.sigmoid(z3)
    return raw / jnp.sum(raw, axis=1, keepdims=True)


def _kernel_body(llm_ref, slm_ref, w1t_ref, b1_ref, w2t_ref, b2_ref,
                 w3t_ref, b3_ref, out_ref, tl_ref, ts_ref, til_ref, tis_ref):
    g = pl.program_id(0)
    c = pl.program_id(1)

    @pl.when(c == 0)
    def _():
        neg = jnp.full((_RG, _K * _LANES), _NEG, dtype=jnp.float32)
        til_ref[...] = neg
        tis_ref[...] = neg

    _insert_chunk(llm_ref, til_ref)
    _insert_chunk(slm_ref, tis_ref)

    @pl.when(c == _NC - 1)
    def _():
        tl_ref[pl.ds(g * _RG, _RG), :] = _merge_topk(til_ref)
        ts_ref[pl.ds(g * _RG, _RG), :] = _merge_topk(tis_ref)

    @pl.when((g == _NG - 1) & (c == _NC - 1))
    def _():
        cc = jnp.concatenate([tl_ref[...], ts_ref[...]], axis=1)
        out_ref[...] = _mlp(cc, w1t_ref, b1_ref, w2t_ref, b2_ref,
                            w3t_ref, b3_ref)


def kernel(llm_logits, slm_logits, W1, b1, W2, b2, W3, b3):
    llm32 = llm_logits.astype(jnp.float32)
    slm32 = slm_logits.astype(jnp.float32)
    w1t = W1.T.astype(jnp.float32)
    w2t = W2.T.astype(jnp.float32)
    w3t = W3.T.astype(jnp.float32)
    b1r = b1.reshape(1, -1).astype(jnp.float32)
    b2r = b2.reshape(1, -1).astype(jnp.float32)
    b3r = b3.reshape(1, -1).astype(jnp.float32)

    full = lambda shape: pl.BlockSpec(shape, lambda g, c: (0,) * len(shape))
    out = pl.pallas_call(
        _kernel_body,
        grid=(_NG, _NC),
        in_specs=[
            pl.BlockSpec((_RG, _CHUNK), lambda g, c: (g, c)),
            pl.BlockSpec((_RG, _CHUNK), lambda g, c: (g, c)),
            full(w1t.shape), full(b1r.shape),
            full(w2t.shape), full(b2r.shape),
            full(w3t.shape), full(b3r.shape),
        ],
        out_specs=pl.BlockSpec((_B, 2), lambda g, c: (0, 0)),
        out_shape=jax.ShapeDtypeStruct((_B, 2), jnp.float32),
        scratch_shapes=[
            pltpu.VMEM((_B, _K), jnp.float32),
            pltpu.VMEM((_B, _K), jnp.float32),
            pltpu.VMEM((_RG, _K * _LANES), jnp.float32),
            pltpu.VMEM((_RG, _K * _LANES), jnp.float32),
        ],
    )(llm32, slm32, w1t, b1r, w2t, b2r, w3t, b3r)
    return out.astype(jnp.float16)
